# baseline (speedup 1.0000x reference)
import functools

import jax
import jax.numpy as jnp
from jax import lax
from jax.experimental import pallas as pl
from jax.experimental.pallas import tpu as pltpu


def _table_kernel(win0_ref, wbig_ref, bias_ref, eye_ref, lab_ref, labt_ref,
                  lnorm_ref, arg_ref, pred_ref, *, M, C, T_l, Jp):
    F = T_l * Jp
    wbig = wbig_ref[...]
    bias = bias_ref[...]
    lab = lab_ref[...]
    labt = labt_ref[...]
    lane_t = lax.broadcasted_iota(jnp.int32, (1, F), 1) // Jp
    cidx1 = lax.broadcasted_iota(jnp.int32, (1, C), 1)
    cidxM = lax.broadcasted_iota(jnp.int32, (M, C), 1)

    def project(win):
        return jnp.dot(win, wbig, preferred_element_type=jnp.float32) + bias

    def classify(pred, t):
        p = jnp.where(lane_t < t, pred, 0.0)
        pdotl = jnp.dot(p, labt, preferred_element_type=jnp.float32)
        score = lnorm_ref[t - 1] - 2.0 * pdotl
        minv = jnp.min(score, axis=-1, keepdims=True)
        return jnp.min(jnp.where(score == minv, cidxM, C),
                       axis=-1, keepdims=True)

    arg0 = classify(project(win0_ref[...]), 1)[0:1, :]

    win_lab = jnp.dot(eye_ref[...], lab,
                      preferred_element_type=jnp.float32)
    pred_table = project(win_lab)

    onehots = []
    for t in range(1, T_l + 1):
        arg = classify(pred_table, t)[0:C, :]
        onehots.append(
            (cidxM[0:C, :] == arg).astype(jnp.float32))

    oh = (cidx1 == arg0).astype(jnp.float32)
    for s in range(1, T_l):
        oh = jnp.dot(oh, onehots[s - 1], preferred_element_type=jnp.float32)

    pred_fin = jnp.dot(oh, pred_table[0:C, :],
                       preferred_element_type=jnp.float32)
    oh_fin = jnp.dot(oh, onehots[T_l - 1], preferred_element_type=jnp.float32)
    arg_fin = jnp.min(jnp.where(oh_fin > 0.5, cidx1, C),
                      axis=-1, keepdims=True)
    arg_ref[...] = jnp.broadcast_to(arg_fin, arg_ref.shape)
    pred_ref[...] = pred_fin


def kernel(x, lens, W, b, sos, label_seqs):
    B = x.shape[0]
    C, T_l, J = label_seqs.shape
    Jp = max(128, ((J + 127) // 128) * 128)
    F = T_l * Jp
    M = max(B // 2 if (B >= 16 and B % 16 == 0) else B, C)

    pad3 = ((0, 0), (0, 0), (0, Jp - J))
    lab = jnp.pad(label_seqs.astype(jnp.float32), pad3).reshape(C, F)
    labt = lab.T
    W_p = jnp.pad(W.astype(jnp.float32), ((0, Jp - J), (0, Jp - J)))
    wbig = jnp.kron(jnp.eye(T_l, dtype=jnp.float32), W_p)
    bias = jnp.tile(jnp.pad(b.astype(jnp.float32), (0, Jp - J)),
                    (T_l,)).reshape(1, F)
    lnorm = jnp.cumsum(jnp.sum(label_seqs.astype(jnp.float32) ** 2, axis=-1),
                       axis=-1).T.reshape(T_l, 1, C)
    win0 = jnp.zeros((M, F), jnp.float32).at[0, :J].set(
        sos.astype(jnp.float32))
    eye_sel = jnp.eye(M, C, dtype=jnp.float32)

    kern = functools.partial(_table_kernel, M=M, C=C, T_l=T_l, Jp=Jp)
    arg_out, pred_out = pl.pallas_call(
        kern,
        out_shape=(jax.ShapeDtypeStruct((1, C), jnp.int32),
                   jax.ShapeDtypeStruct((1, F), jnp.float32)),
        grid=(1,),
        in_specs=[
            pl.BlockSpec((M, F), lambda i: (0, 0)),
            pl.BlockSpec((F, F), lambda i: (0, 0)),
            pl.BlockSpec((1, F), lambda i: (0, 0)),
            pl.BlockSpec((M, C), lambda i: (0, 0)),
            pl.BlockSpec((C, F), lambda i: (0, 0)),
            pl.BlockSpec((F, C), lambda i: (0, 0)),
            pl.BlockSpec((T_l, 1, C), lambda i: (0, 0, 0)),
        ],
        out_specs=(pl.BlockSpec((1, C), lambda i: (0, 0)),
                   pl.BlockSpec((1, F), lambda i: (0, 0))),
        compiler_params=pltpu.CompilerParams(
            dimension_semantics=("arbitrary",)),
    )(win0, wbig, bias, eye_sel, lab, labt, lnorm)

    pred_label_sofar = jnp.broadcast_to(arg_out[0, 0], (B,))
    pred_label_seq = jnp.broadcast_to(
        pred_out.reshape(1, T_l, Jp)[:, :, :J], (B, T_l, J))
    return pred_label_sofar, pred_label_seq


# speedup vs baseline: 2.2128x; 2.2128x over previous
import functools

import jax
import jax.numpy as jnp
from jax import lax
from jax.experimental import pallas as pl
from jax.experimental.pallas import tpu as pltpu


def _table_kernel(w_ref, b_ref, sos_ref, lab_ref, labt_ref, arg_ref, pred_ref,
                  *, C, T_l, J):
    F = T_l * J
    W = w_ref[...]
    b = b_ref[...]
    sos = sos_ref[...]
    lab = lab_ref[...]
    labt = labt_ref[...]
    lane_t = lax.broadcasted_iota(jnp.int32, (1, F), 1) // J
    cidx1 = lax.broadcasted_iota(jnp.int32, (1, C), 1)
    cidxC = lax.broadcasted_iota(jnp.int32, (C, C), 1)
    ccol = lax.broadcasted_iota(jnp.int32, (C, 1), 0)

    pred_table = jnp.concatenate(
        [jnp.dot(lab[:, t * J:(t + 1) * J], W,
                 preferred_element_type=jnp.float32) + b
         for t in range(T_l)], axis=1)

    lnorms = []
    acc = jnp.zeros((1, C), jnp.float32)
    for t in range(T_l):
        sl = labt[t * J:(t + 1) * J, :]
        acc = acc + jnp.sum(sl * sl, axis=0, keepdims=True)
        lnorms.append(acc)

    def classify(p, t, cidx):
        pm = jnp.where(lane_t < t, p, 0.0)
        pd = jnp.dot(pm, labt, preferred_element_type=jnp.float32)
        score = lnorms[t - 1] - 2.0 * pd
        minv = jnp.min(score, axis=-1, keepdims=True)
        return jnp.min(jnp.where(score == minv, cidx, C),
                       axis=-1, keepdims=True)

    onehots = []
    for t in range(1, T_l + 1):
        arg = classify(pred_table, t, cidxC)
        onehots.append((cidxC == arg).astype(jnp.float32))

    p0row = jnp.dot(sos, W, preferred_element_type=jnp.float32) + b
    p0 = jnp.concatenate(
        [p0row] + [jnp.zeros((1, J), jnp.float32)] * (T_l - 1), axis=1)
    arg0 = classify(p0, 1, cidx1)
    oh = (cidx1 == arg0).astype(jnp.float32)

    for s in range(1, T_l):
        oh = jnp.dot(oh, onehots[s - 1], preferred_element_type=jnp.float32)

    oh_fin = jnp.dot(oh, onehots[T_l - 1], preferred_element_type=jnp.float32)
    arg_fin = jnp.min(jnp.where(oh_fin > 0.5, cidx1, C),
                      axis=-1, keepdims=True)
    arg_ref[...] = jnp.broadcast_to(arg_fin, arg_ref.shape)

    arg7 = jnp.min(jnp.where(oh > 0.5, cidx1, C),
                   axis=-1, keepdims=True)
    rowmask = (ccol == arg7).astype(jnp.float32)
    pred_ref[...] = jnp.sum(pred_table * rowmask, axis=0, keepdims=True)


def kernel(x, lens, W, b, sos, label_seqs):
    B = x.shape[0]
    C, T_l, J = label_seqs.shape
    F = T_l * J

    lab = label_seqs.astype(jnp.float32).reshape(C, F)
    labt = lab.T
    b2 = b.astype(jnp.float32).reshape(1, J)
    sos2 = sos.astype(jnp.float32).reshape(1, J)
    Wf = W.astype(jnp.float32)

    kern = functools.partial(_table_kernel, C=C, T_l=T_l, J=J)
    arg_out, pred_out = pl.pallas_call(
        kern,
        out_shape=(jax.ShapeDtypeStruct((1, C), jnp.int32),
                   jax.ShapeDtypeStruct((1, F), jnp.float32)),
        grid=(1,),
        in_specs=[
            pl.BlockSpec((J, J), lambda i: (0, 0)),
            pl.BlockSpec((1, J), lambda i: (0, 0)),
            pl.BlockSpec((1, J), lambda i: (0, 0)),
            pl.BlockSpec((C, F), lambda i: (0, 0)),
            pl.BlockSpec((F, C), lambda i: (0, 0)),
        ],
        out_specs=(pl.BlockSpec((1, C), lambda i: (0, 0)),
                   pl.BlockSpec((1, F), lambda i: (0, 0))),
        compiler_params=pltpu.CompilerParams(
            dimension_semantics=("arbitrary",)),
    )(Wf, b2, sos2, lab, labt)

    pred_label_sofar = jnp.broadcast_to(arg_out[0, 0], (B,))
    pred_label_seq = jnp.broadcast_to(pred_out.reshape(1, T_l, J), (B, T_l, J))
    return pred_label_sofar, pred_label_seq
